# Initial kernel scaffold; baseline (speedup 1.0000x reference)
import jax
import jax.numpy as jnp
from jax.experimental import pallas as pl


def kernel(feature, centroid_ids):
    raise NotImplementedError("write your pallas kernel here")


# speedup vs baseline: 3.8395x; 3.8395x over previous
import functools

import jax
import jax.numpy as jnp
from jax import lax
from jax.experimental import pallas as pl
from jax.experimental.pallas import tpu as pltpu
from jax.experimental.pallas import tpu_sc as plsc

EPS_ = 1e-06
N_ = 4096
D_ = 256
C_ = 128
BLK_ = 512
GRID_ = N_ // BLK_


_SC_WORKERS = 16
_ROWS_PER_W = C_ // _SC_WORKERS


@functools.cache
def _make_sc_gather():
    mesh = plsc.VectorSubcoreMesh(core_axis_name="c", subcore_axis_name="s")

    @functools.partial(
        pl.kernel,
        mesh=mesh,
        out_type=jax.ShapeDtypeStruct((C_, D_), jnp.float32),
        scratch_types=[
            pltpu.VMEM((_ROWS_PER_W,), jnp.int32),
            pltpu.VMEM((_ROWS_PER_W, D_), jnp.float32),
            pltpu.SemaphoreType.DMA,
        ],
    )
    def sc_gather(table_hbm, idx_hbm, out_hbm, idx_v, rows_v, sem):
        wid = lax.axis_index("s") * 2 + lax.axis_index("c")

        @pl.when(wid < _SC_WORKERS)
        def _():
            base = wid * _ROWS_PER_W
            pltpu.sync_copy(idx_hbm.at[pl.ds(base, _ROWS_PER_W)], idx_v)
            pltpu.async_copy(table_hbm.at[idx_v], rows_v, sem).wait()
            pltpu.sync_copy(rows_v, out_hbm.at[pl.ds(base, _ROWS_PER_W)])

    return sc_gather


def _tc_body(f_ref, cf_ref, cidf_ref, csq_ref, cs_ref, e_ref, y_ref):
    i = pl.program_id(0)
    f = f_ref[...]
    cf = cf_ref[...]
    cidf = cidf_ref[...]
    csq = csq_ref[...]
    cs = cs_ref[...]

    dn = (((1,), (1,)), ((), ()))
    sq = jnp.sum(f * f, axis=1, keepdims=True)
    s = jnp.sum(f, axis=1, keepdims=True)
    g = lax.dot_general(f, cf, dn,
                        preferred_element_type=jnp.float32)

    d2 = sq + csq - 2.0 * g + 2.0 * EPS_ * (s - cs) + D_ * EPS_ * EPS_
    dist = jnp.sqrt(jnp.maximum(d2, 0.0))

    m = jnp.min(dist, axis=1, keepdims=True)
    jidx = lax.broadcasted_iota(jnp.int32, (BLK_, C_), 1).astype(jnp.float32)
    pred = jnp.min(jnp.where(dist == m, jidx, float(C_)),
                   axis=1, keepdims=True)

    rowid = ((i * BLK_).astype(jnp.float32)
             + lax.broadcasted_iota(jnp.int32, (BLK_, 1), 0)
               .astype(jnp.float32))
    eq = cidf == rowid
    mask = jnp.max(jnp.where(eq, 1.0, 0.0), axis=1, keepdims=True)
    cvect = jnp.sum(jnp.where(eq, jidx, 0.0), axis=1, keepdims=True)

    y_ref[...] = (1.0 - mask) * pred + cvect

    @pl.when(i == 0)
    def _():
        e_ref[...] = jnp.zeros((1, 1), jnp.float32)

    e_ref[...] = e_ref[...] - jnp.sum(m, keepdims=True).reshape(1, 1)


_tc_call = pl.pallas_call(
    _tc_body,
    grid=(GRID_,),
    in_specs=[
        pl.BlockSpec((BLK_, D_), lambda i: (i, 0)),
        pl.BlockSpec((C_, D_), lambda i: (0, 0)),
        pl.BlockSpec((1, C_), lambda i: (0, 0)),
        pl.BlockSpec((1, C_), lambda i: (0, 0)),
        pl.BlockSpec((1, C_), lambda i: (0, 0)),
    ],
    out_specs=[
        pl.BlockSpec((1, 1), lambda i: (0, 0)),
        pl.BlockSpec((BLK_, 1), lambda i: (i, 0)),
    ],
    out_shape=[
        jax.ShapeDtypeStruct((1, 1), jnp.float32),
        jax.ShapeDtypeStruct((N_, 1), jnp.float32),
    ],
    compiler_params=pltpu.CompilerParams(
        dimension_semantics=("arbitrary",),
    ),
)


def kernel(feature, centroid_ids):
    cfeat = _make_sc_gather()(feature, centroid_ids)
    cidf = centroid_ids.astype(jnp.float32).reshape(1, C_)
    csq = jnp.sum(cfeat * cfeat, axis=1).reshape(1, C_)
    cs = jnp.sum(cfeat, axis=1).reshape(1, C_)
    e, y = _tc_call(feature, cfeat, cidf, csq, cs)
    return e.reshape(()), y.reshape(N_)


# speedup vs baseline: 3.9973x; 1.0411x over previous
import functools

import jax
import jax.numpy as jnp
from jax import lax
from jax.experimental import pallas as pl
from jax.experimental.pallas import tpu as pltpu
from jax.experimental.pallas import tpu_sc as plsc

EPS_ = 1e-06
N_ = 4096
D_ = 256
C_ = 128
BLK_ = 512
GRID_ = N_ // BLK_


_SC_WORKERS = 16
_ROWS_PER_W = C_ // _SC_WORKERS


@functools.cache
def _make_sc_gather():
    mesh = plsc.VectorSubcoreMesh(core_axis_name="c", subcore_axis_name="s")

    @functools.partial(
        pl.kernel,
        mesh=mesh,
        out_type=jax.ShapeDtypeStruct((C_, D_), jnp.float32),
        scratch_types=[
            pltpu.VMEM((_ROWS_PER_W,), jnp.int32),
            pltpu.VMEM((_ROWS_PER_W, D_), jnp.float32),
            pltpu.SemaphoreType.DMA,
        ],
    )
    def sc_gather(table_hbm, idx_hbm, out_hbm, idx_v, rows_v, sem):
        wid = lax.axis_index("s") * 2 + lax.axis_index("c")

        @pl.when(wid < _SC_WORKERS)
        def _():
            base = wid * _ROWS_PER_W
            pltpu.sync_copy(idx_hbm.at[pl.ds(base, _ROWS_PER_W)], idx_v)
            pltpu.async_copy(table_hbm.at[idx_v], rows_v, sem).wait()
            pltpu.sync_copy(rows_v, out_hbm.at[pl.ds(base, _ROWS_PER_W)])

    return sc_gather


def _tc_body(f_ref, cf_ref, cid_ref, e_ref, y_ref):
    i = pl.program_id(0)
    f = f_ref[...]
    cf = cf_ref[...]
    cidf = cid_ref[...].astype(jnp.float32)

    dn = (((1,), (1,)), ((), ()))
    csq = jnp.sum(cf * cf, axis=1, keepdims=True).reshape(1, C_)
    cs = jnp.sum(cf, axis=1, keepdims=True).reshape(1, C_)
    sq = jnp.sum(f * f, axis=1, keepdims=True)
    s = jnp.sum(f, axis=1, keepdims=True)
    g = lax.dot_general(f, cf, dn,
                        preferred_element_type=jnp.float32)

    d2 = sq + csq - 2.0 * g + 2.0 * EPS_ * (s - cs) + D_ * EPS_ * EPS_
    dist = jnp.sqrt(jnp.maximum(d2, 0.0))

    m = jnp.min(dist, axis=1, keepdims=True)
    jidx = lax.broadcasted_iota(jnp.int32, (BLK_, C_), 1).astype(jnp.float32)
    pred = jnp.min(jnp.where(dist == m, jidx, float(C_)),
                   axis=1, keepdims=True)

    rowid = ((i * BLK_).astype(jnp.float32)
             + lax.broadcasted_iota(jnp.int32, (BLK_, 1), 0)
               .astype(jnp.float32))
    eq = cidf == rowid
    mask = jnp.max(jnp.where(eq, 1.0, 0.0), axis=1, keepdims=True)
    cvect = jnp.sum(jnp.where(eq, jidx, 0.0), axis=1, keepdims=True)

    y_ref[...] = (1.0 - mask) * pred + cvect

    @pl.when(i == 0)
    def _():
        e_ref[...] = jnp.zeros((1, 1), jnp.float32)

    e_ref[...] = e_ref[...] - jnp.sum(m, keepdims=True).reshape(1, 1)


_tc_call = pl.pallas_call(
    _tc_body,
    grid=(GRID_,),
    in_specs=[
        pl.BlockSpec((BLK_, D_), lambda i: (i, 0)),
        pl.BlockSpec((C_, D_), lambda i: (0, 0)),
        pl.BlockSpec((1, C_), lambda i: (0, 0)),
    ],
    out_specs=[
        pl.BlockSpec((1, 1), lambda i: (0, 0)),
        pl.BlockSpec((BLK_, 1), lambda i: (i, 0)),
    ],
    out_shape=[
        jax.ShapeDtypeStruct((1, 1), jnp.float32),
        jax.ShapeDtypeStruct((N_, 1), jnp.float32),
    ],
    compiler_params=pltpu.CompilerParams(
        dimension_semantics=("arbitrary",),
    ),
)


def kernel(feature, centroid_ids):
    cfeat = _make_sc_gather()(feature, centroid_ids)
    e, y = _tc_call(feature, cfeat, centroid_ids.reshape(1, C_))
    return e.reshape(()), y.reshape(N_)


# speedup vs baseline: 4.4252x; 1.1070x over previous
import functools

import jax
import jax.numpy as jnp
from jax import lax
from jax.experimental import pallas as pl
from jax.experimental.pallas import tpu as pltpu
from jax.experimental.pallas import tpu_sc as plsc

EPS_ = 1e-06
N_ = 4096
D_ = 256
C_ = 128
BLK_ = 512
GRID_ = N_ // BLK_


_SC_WORKERS = 16
_ROWS_PER_W = C_ // _SC_WORKERS


@functools.cache
def _make_sc_gather():
    mesh = plsc.VectorSubcoreMesh(core_axis_name="c", subcore_axis_name="s")

    @functools.partial(
        pl.kernel,
        mesh=mesh,
        out_type=jax.ShapeDtypeStruct((C_, D_), jnp.float32),
        scratch_types=[
            pltpu.VMEM((_ROWS_PER_W,), jnp.int32),
            pltpu.VMEM((_ROWS_PER_W, D_), jnp.float32),
            pltpu.SemaphoreType.DMA,
        ],
    )
    def sc_gather(table_hbm, idx_hbm, out_hbm, idx_v, rows_v, sem):
        wid = lax.axis_index("s") * 2 + lax.axis_index("c")

        @pl.when(wid < _SC_WORKERS)
        def _():
            base = wid * _ROWS_PER_W
            pltpu.sync_copy(idx_hbm.at[pl.ds(base, _ROWS_PER_W)], idx_v)
            pltpu.async_copy(table_hbm.at[idx_v], rows_v, sem).wait()
            pltpu.sync_copy(rows_v, out_hbm.at[pl.ds(base, _ROWS_PER_W)])

    return sc_gather


def _tc_body(f_ref, cf_ref, cid_ref, e_ref, y_ref):
    i = pl.program_id(0)
    f = f_ref[...]
    cf = cf_ref[...]
    cidf = cid_ref[...].astype(jnp.float32)

    dn = (((1,), (1,)), ((), ()))
    csq = jnp.sum(cf * cf, axis=1, keepdims=True).reshape(1, C_)
    cs = jnp.sum(cf, axis=1, keepdims=True).reshape(1, C_)
    sq = jnp.sum(f * f, axis=1, keepdims=True)
    s = jnp.sum(f, axis=1, keepdims=True)
    g = lax.dot_general(f, cf, dn,
                        preferred_element_type=jnp.float32)

    d2 = sq + csq - 2.0 * g + 2.0 * EPS_ * (s - cs) + D_ * EPS_ * EPS_
    dist = jnp.sqrt(jnp.maximum(d2, 0.0))

    m = jnp.min(dist, axis=1, keepdims=True)
    jidx = lax.broadcasted_iota(jnp.int32, (BLK_, C_), 1).astype(jnp.float32)
    pred = jnp.min(jnp.where(dist == m, jidx, float(C_)),
                   axis=1, keepdims=True)

    rowid = ((i * BLK_).astype(jnp.float32)
             + lax.broadcasted_iota(jnp.int32, (BLK_, 1), 0)
               .astype(jnp.float32))
    eq = cidf == rowid
    mask = jnp.max(jnp.where(eq, 1.0, 0.0), axis=1, keepdims=True)
    cvect = jnp.sum(jnp.where(eq, jidx, 0.0), axis=1, keepdims=True)

    y = (1.0 - mask) * pred + cvect
    y_ref[...] = y.reshape(1, BLK_ // 128, 128)

    @pl.when(i == 0)
    def _():
        e_ref[...] = jnp.zeros((1, 1), jnp.float32)

    e_ref[...] = e_ref[...] - jnp.sum(m, keepdims=True).reshape(1, 1)


_tc_call = pl.pallas_call(
    _tc_body,
    grid=(GRID_,),
    in_specs=[
        pl.BlockSpec((BLK_, D_), lambda i: (i, 0)),
        pl.BlockSpec((C_, D_), lambda i: (0, 0)),
        pl.BlockSpec((1, C_), lambda i: (0, 0)),
    ],
    out_specs=[
        pl.BlockSpec((1, 1), lambda i: (0, 0)),
        pl.BlockSpec((1, BLK_ // 128, 128), lambda i: (i, 0, 0)),
    ],
    out_shape=[
        jax.ShapeDtypeStruct((1, 1), jnp.float32),
        jax.ShapeDtypeStruct((GRID_, BLK_ // 128, 128), jnp.float32),
    ],
    compiler_params=pltpu.CompilerParams(
        dimension_semantics=("arbitrary",),
    ),
)


def kernel(feature, centroid_ids):
    cfeat = _make_sc_gather()(feature, centroid_ids)
    e, y = _tc_call(feature, cfeat, centroid_ids.reshape(1, C_))
    return e.reshape(()), y.reshape(N_)


# speedup vs baseline: 4.7735x; 1.0787x over previous
import functools

import jax
import jax.numpy as jnp
from jax import lax
from jax.experimental import pallas as pl
from jax.experimental.pallas import tpu as pltpu
from jax.experimental.pallas import tpu_sc as plsc

EPS_ = 1e-06
N_ = 4096
D_ = 256
C_ = 128
BLK_ = 1024
GRID_ = N_ // BLK_


_SC_WORKERS = 16
_ROWS_PER_W = C_ // _SC_WORKERS


@functools.cache
def _make_sc_gather():
    mesh = plsc.VectorSubcoreMesh(core_axis_name="c", subcore_axis_name="s")

    @functools.partial(
        pl.kernel,
        mesh=mesh,
        out_type=jax.ShapeDtypeStruct((C_, D_), jnp.float32),
        scratch_types=[
            pltpu.VMEM((_ROWS_PER_W,), jnp.int32),
            pltpu.VMEM((_ROWS_PER_W, D_), jnp.float32),
            pltpu.SemaphoreType.DMA,
        ],
    )
    def sc_gather(table_hbm, idx_hbm, out_hbm, idx_v, rows_v, sem):
        wid = lax.axis_index("s") * 2 + lax.axis_index("c")

        @pl.when(wid < _SC_WORKERS)
        def _():
            base = wid * _ROWS_PER_W
            pltpu.sync_copy(idx_hbm.at[pl.ds(base, _ROWS_PER_W)], idx_v)
            pltpu.async_copy(table_hbm.at[idx_v], rows_v, sem).wait()
            pltpu.sync_copy(rows_v, out_hbm.at[pl.ds(base, _ROWS_PER_W)])

    return sc_gather


def _tc_body(f_ref, cf_ref, cid_ref, e_ref, y_ref):
    i = pl.program_id(0)
    f = f_ref[...]
    cf = cf_ref[...]
    cidf = cid_ref[...].astype(jnp.float32)

    dn = (((1,), (1,)), ((), ()))
    csq = jnp.sum(cf * cf, axis=1, keepdims=True).reshape(1, C_)
    cs = jnp.sum(cf, axis=1, keepdims=True).reshape(1, C_)
    sq = jnp.sum(f * f, axis=1, keepdims=True)
    s = jnp.sum(f, axis=1, keepdims=True)
    g = lax.dot_general(f, cf, dn,
                        preferred_element_type=jnp.float32)

    d2 = sq + csq - 2.0 * g + 2.0 * EPS_ * (s - cs) + D_ * EPS_ * EPS_
    dist = jnp.sqrt(jnp.maximum(d2, 0.0))

    m = jnp.min(dist, axis=1, keepdims=True)
    jidx = lax.broadcasted_iota(jnp.int32, (BLK_, C_), 1).astype(jnp.float32)
    pred = jnp.min(jnp.where(dist == m, jidx, float(C_)),
                   axis=1, keepdims=True)

    rowid = ((i * BLK_).astype(jnp.float32)
             + lax.broadcasted_iota(jnp.int32, (BLK_, 1), 0)
               .astype(jnp.float32))
    eq = cidf == rowid
    mask = jnp.max(jnp.where(eq, 1.0, 0.0), axis=1, keepdims=True)
    cvect = jnp.sum(jnp.where(eq, jidx, 0.0), axis=1, keepdims=True)

    y = (1.0 - mask) * pred + cvect
    y_ref[...] = y.reshape(1, BLK_ // 128, 128)

    @pl.when(i == 0)
    def _():
        e_ref[...] = jnp.zeros((1, 1), jnp.float32)

    e_ref[...] = e_ref[...] - jnp.sum(m, keepdims=True).reshape(1, 1)


_tc_call = pl.pallas_call(
    _tc_body,
    grid=(GRID_,),
    in_specs=[
        pl.BlockSpec((BLK_, D_), lambda i: (i, 0)),
        pl.BlockSpec((C_, D_), lambda i: (0, 0)),
        pl.BlockSpec((1, C_), lambda i: (0, 0)),
    ],
    out_specs=[
        pl.BlockSpec((1, 1), lambda i: (0, 0)),
        pl.BlockSpec((1, BLK_ // 128, 128), lambda i: (i, 0, 0)),
    ],
    out_shape=[
        jax.ShapeDtypeStruct((1, 1), jnp.float32),
        jax.ShapeDtypeStruct((GRID_, BLK_ // 128, 128), jnp.float32),
    ],
    compiler_params=pltpu.CompilerParams(
        dimension_semantics=("arbitrary",),
    ),
)


def kernel(feature, centroid_ids):
    cfeat = _make_sc_gather()(feature, centroid_ids)
    e, y = _tc_call(feature, cfeat, centroid_ids.reshape(1, C_))
    return e.reshape(()), y.reshape(N_)


# speedup vs baseline: 4.8693x; 1.0201x over previous
import functools

import jax
import jax.numpy as jnp
from jax import lax
from jax.experimental import pallas as pl
from jax.experimental.pallas import tpu as pltpu
from jax.experimental.pallas import tpu_sc as plsc

EPS_ = 1e-06
N_ = 4096
D_ = 256
C_ = 128
BLK_ = 1024
GRID_ = N_ // BLK_


_SC_WORKERS = 16
_ROWS_PER_W = C_ // _SC_WORKERS


@functools.cache
def _make_sc_gather():
    mesh = plsc.VectorSubcoreMesh(core_axis_name="c", subcore_axis_name="s")

    @functools.partial(
        pl.kernel,
        mesh=mesh,
        out_type=jax.ShapeDtypeStruct((C_, D_), jnp.float32),
        scratch_types=[
            pltpu.VMEM((_ROWS_PER_W,), jnp.int32),
            pltpu.VMEM((_ROWS_PER_W, D_), jnp.float32),
            pltpu.SemaphoreType.DMA,
        ],
    )
    def sc_gather(table_hbm, idx_hbm, out_hbm, idx_v, rows_v, sem):
        wid = lax.axis_index("s") * 2 + lax.axis_index("c")

        @pl.when(wid < _SC_WORKERS)
        def _():
            base = wid * _ROWS_PER_W
            pltpu.sync_copy(idx_hbm.at[pl.ds(base, _ROWS_PER_W)], idx_v)
            pltpu.async_copy(table_hbm.at[idx_v], rows_v, sem).wait()
            pltpu.sync_copy(rows_v, out_hbm.at[pl.ds(base, _ROWS_PER_W)])

    return sc_gather


def _tc_body(f_ref, cf_ref, cid_ref, e_ref, y_ref, csq_ref, cs_ref):
    i = pl.program_id(0)
    f = f_ref[...]
    cf = cf_ref[...]
    cidf = cid_ref[...].astype(jnp.float32)

    dn = (((1,), (1,)), ((), ()))

    @pl.when(i == 0)
    def _():
        csq_ref[...] = jnp.sum(cf * cf, axis=1, keepdims=True).reshape(1, C_)
        cs_ref[...] = jnp.sum(cf, axis=1, keepdims=True).reshape(1, C_)
        e_ref[...] = jnp.zeros((1, 1), jnp.float32)

    csq = csq_ref[...]
    cs = cs_ref[...]
    sq = jnp.sum(f * f, axis=1, keepdims=True)
    s = jnp.sum(f, axis=1, keepdims=True)
    g = lax.dot_general(f, cf, dn,
                        preferred_element_type=jnp.float32)

    d2 = sq + csq - 2.0 * g + 2.0 * EPS_ * (s - cs) + D_ * EPS_ * EPS_
    dist = jnp.sqrt(jnp.maximum(d2, 0.0))

    m = jnp.min(dist, axis=1, keepdims=True)
    jidx = lax.broadcasted_iota(jnp.int32, (BLK_, C_), 1).astype(jnp.float32)
    pred = jnp.min(jnp.where(dist == m, jidx, float(C_)),
                   axis=1, keepdims=True)

    rowid = ((i * BLK_).astype(jnp.float32)
             + lax.broadcasted_iota(jnp.int32, (BLK_, 1), 0)
               .astype(jnp.float32))
    eq = cidf == rowid
    mask = jnp.max(jnp.where(eq, 1.0, 0.0), axis=1, keepdims=True)
    cvect = jnp.sum(jnp.where(eq, jidx, 0.0), axis=1, keepdims=True)

    y = (1.0 - mask) * pred + cvect
    y_ref[...] = y.reshape(1, BLK_ // 128, 128)

    e_blk = lax.dot_general(jnp.ones((1, BLK_), jnp.float32), m,
                            (((1,), (0,)), ((), ())),
                            preferred_element_type=jnp.float32)
    e_ref[...] = e_ref[...] - e_blk


_tc_call = pl.pallas_call(
    _tc_body,
    grid=(GRID_,),
    in_specs=[
        pl.BlockSpec((BLK_, D_), lambda i: (i, 0)),
        pl.BlockSpec((C_, D_), lambda i: (0, 0)),
        pl.BlockSpec((1, C_), lambda i: (0, 0)),
    ],
    out_specs=[
        pl.BlockSpec((1, 1), lambda i: (0, 0)),
        pl.BlockSpec((1, BLK_ // 128, 128), lambda i: (i, 0, 0)),
    ],
    out_shape=[
        jax.ShapeDtypeStruct((1, 1), jnp.float32),
        jax.ShapeDtypeStruct((GRID_, BLK_ // 128, 128), jnp.float32),
    ],
    scratch_shapes=[
        pltpu.VMEM((1, C_), jnp.float32),
        pltpu.VMEM((1, C_), jnp.float32),
    ],
    compiler_params=pltpu.CompilerParams(
        dimension_semantics=("arbitrary",),
    ),
)


def kernel(feature, centroid_ids):
    cfeat = _make_sc_gather()(feature, centroid_ids)
    e, y = _tc_call(feature, cfeat, centroid_ids.reshape(1, C_))
    return e.reshape(()), y.reshape(N_)


# speedup vs baseline: 4.9056x; 1.0074x over previous
import functools

import jax
import jax.numpy as jnp
from jax import lax
from jax.experimental import pallas as pl
from jax.experimental.pallas import tpu as pltpu
from jax.experimental.pallas import tpu_sc as plsc

EPS_ = 1e-06
N_ = 4096
D_ = 256
C_ = 128
BLK_ = 2048
GRID_ = N_ // BLK_


_SC_WORKERS = 16
_ROWS_PER_W = C_ // _SC_WORKERS


@functools.cache
def _make_sc_gather():
    mesh = plsc.VectorSubcoreMesh(core_axis_name="c", subcore_axis_name="s")

    @functools.partial(
        pl.kernel,
        mesh=mesh,
        out_type=jax.ShapeDtypeStruct((C_, D_), jnp.float32),
        scratch_types=[
            pltpu.VMEM((_ROWS_PER_W,), jnp.int32),
            pltpu.VMEM((_ROWS_PER_W, D_), jnp.float32),
            pltpu.SemaphoreType.DMA,
        ],
    )
    def sc_gather(table_hbm, idx_hbm, out_hbm, idx_v, rows_v, sem):
        wid = lax.axis_index("s") * 2 + lax.axis_index("c")

        @pl.when(wid < _SC_WORKERS)
        def _():
            base = wid * _ROWS_PER_W
            pltpu.sync_copy(idx_hbm.at[pl.ds(base, _ROWS_PER_W)], idx_v)
            pltpu.async_copy(table_hbm.at[idx_v], rows_v, sem).wait()
            pltpu.sync_copy(rows_v, out_hbm.at[pl.ds(base, _ROWS_PER_W)])

    return sc_gather


def _tc_body(f_ref, cf_ref, cid_ref, e_ref, y_ref, csq_ref, cs_ref):
    i = pl.program_id(0)
    f = f_ref[...]
    cf = cf_ref[...]
    cidf = cid_ref[...].astype(jnp.float32)

    dn = (((1,), (1,)), ((), ()))

    @pl.when(i == 0)
    def _():
        csq_ref[...] = jnp.sum(cf * cf, axis=1, keepdims=True).reshape(1, C_)
        cs_ref[...] = jnp.sum(cf, axis=1, keepdims=True).reshape(1, C_)
        e_ref[...] = jnp.zeros((1, 1), jnp.float32)

    csq = csq_ref[...]
    cs = cs_ref[...]
    sq = jnp.sum(f * f, axis=1, keepdims=True)
    s = jnp.sum(f, axis=1, keepdims=True)
    g = lax.dot_general(f, cf, dn,
                        preferred_element_type=jnp.float32)

    d2 = sq + csq - 2.0 * g + 2.0 * EPS_ * (s - cs) + D_ * EPS_ * EPS_
    dist = jnp.sqrt(jnp.maximum(d2, 0.0))

    m = jnp.min(dist, axis=1, keepdims=True)
    jidx = lax.broadcasted_iota(jnp.int32, (BLK_, C_), 1).astype(jnp.float32)
    pred = jnp.min(jnp.where(dist == m, jidx, float(C_)),
                   axis=1, keepdims=True)

    rowid = ((i * BLK_).astype(jnp.float32)
             + lax.broadcasted_iota(jnp.int32, (BLK_, 1), 0)
               .astype(jnp.float32))
    eq = cidf == rowid
    mask = jnp.max(jnp.where(eq, 1.0, 0.0), axis=1, keepdims=True)
    cvect = jnp.sum(jnp.where(eq, jidx, 0.0), axis=1, keepdims=True)

    y = (1.0 - mask) * pred + cvect
    y_ref[...] = y.reshape(1, BLK_ // 128, 128)

    e_blk = lax.dot_general(jnp.ones((1, BLK_), jnp.float32), m,
                            (((1,), (0,)), ((), ())),
                            preferred_element_type=jnp.float32)
    e_ref[...] = e_ref[...] - e_blk


_tc_call = pl.pallas_call(
    _tc_body,
    grid=(GRID_,),
    in_specs=[
        pl.BlockSpec((BLK_, D_), lambda i: (i, 0)),
        pl.BlockSpec((C_, D_), lambda i: (0, 0)),
        pl.BlockSpec((1, C_), lambda i: (0, 0)),
    ],
    out_specs=[
        pl.BlockSpec((1, 1), lambda i: (0, 0)),
        pl.BlockSpec((1, BLK_ // 128, 128), lambda i: (i, 0, 0)),
    ],
    out_shape=[
        jax.ShapeDtypeStruct((1, 1), jnp.float32),
        jax.ShapeDtypeStruct((GRID_, BLK_ // 128, 128), jnp.float32),
    ],
    scratch_shapes=[
        pltpu.VMEM((1, C_), jnp.float32),
        pltpu.VMEM((1, C_), jnp.float32),
    ],
    compiler_params=pltpu.CompilerParams(
        dimension_semantics=("arbitrary",),
    ),
)


def kernel(feature, centroid_ids):
    cfeat = _make_sc_gather()(feature, centroid_ids)
    e, y = _tc_call(feature, cfeat, centroid_ids.reshape(1, C_))
    return e.reshape(()), y.reshape(N_)
